# baseline (speedup 1.0000x reference)
import math

import jax
import jax.numpy as jnp
from jax.experimental import pallas as pl
from jax.experimental.pallas import tpu as pltpu

_K = 8
_BS = 2


def _body(pf_ref, xa_ref, xb_ref, t_ref, sel_ref, ml_ref):
    _, h, w, d = t_ref.shape
    hh = h // 2
    inv_n = 1.0 / (h * w * d)
    penalty = math.log(_K, 2) / (h * w)

    for s in range(_BS):
        ta = t_ref[s, :hh]
        tb = t_ref[s, hh:]

        best_scaled = jnp.float32(jnp.inf)
        best_loss = jnp.float32(0.0)
        best_idx = jnp.int32(0)
        for k in range(_K):
            da = xa_ref[s, :, :, k * d:(k + 1) * d] - ta
            db = xb_ref[s, :, :, k * d:(k + 1) * d] - tb
            loss_k = (jnp.sum(da * da) + jnp.sum(db * db)) * inv_n + penalty
            scaled_k = loss_k * pf_ref[0, k]
            better = scaled_k < best_scaled
            best_scaled = jnp.where(better, scaled_k, best_scaled)
            best_loss = jnp.where(better, loss_k, best_loss)
            best_idx = jnp.where(better, jnp.int32(k), best_idx)

        ml_ref[s] = jnp.full((1, 128), best_loss, jnp.float32)
        for k in range(_K):
            @pl.when(best_idx == k)
            def _():
                sel_ref[s, :hh] = xa_ref[s, :, :, k * d:(k + 1) * d]
                sel_ref[s, hh:] = xb_ref[s, :, :, k * d:(k + 1) * d]


def kernel(x, target, pick_frequency):
    B, C, H, W = x.shape
    D = C // _K
    xt = jnp.transpose(x, (0, 2, 3, 1))
    tt = jnp.transpose(target, (0, 2, 3, 1))
    pf = pick_frequency.reshape(1, _K)

    sel, ml = pl.pallas_call(
        _body,
        grid=(B // _BS,),
        in_specs=[
            pl.BlockSpec(memory_space=pltpu.SMEM),
            pl.BlockSpec((_BS, H // 2, W, C), lambda b: (b, 0, 0, 0)),
            pl.BlockSpec((_BS, H // 2, W, C), lambda b: (b, 1, 0, 0)),
            pl.BlockSpec((_BS, H, W, D), lambda b: (b, 0, 0, 0)),
        ],
        out_specs=[
            pl.BlockSpec((_BS, H, W, D), lambda b: (b, 0, 0, 0)),
            pl.BlockSpec((_BS, 1, 128), lambda b: (b, 0, 0)),
        ],
        out_shape=[
            jax.ShapeDtypeStruct((B, H, W, D), jnp.float32),
            jax.ShapeDtypeStruct((B, 1, 128), jnp.float32),
        ],
        compiler_params=pltpu.CompilerParams(
            dimension_semantics=("parallel",),
        ),
    )(pf, xt, xt, tt)

    selected = jnp.transpose(sel, (0, 3, 1, 2))
    min_loss = ml[:, 0, 0]
    return selected, min_loss
